# baseline (speedup 1.0000x reference)
import jax
import jax.numpy as jnp
from jax.experimental import pallas as pl

U, I, C = 100000, 50000, 1000
NY, NM, ND = 20, 13, 32
D, T, L = 64, 50, 3
B = 1024
BETA = 0.1
NEG = -2.0 ** 32 + 1


def _spmm(rows, cols, vals, mat, n):
    return jax.ops.segment_sum(vals[:, None] * mat[cols], rows, num_segments=n)


def _propagate(rows, cols, vals, ego, Wg, bg, Wb, bb, n):
    embs = [ego]
    for i in range(L):
        side = _spmm(rows, cols, vals, ego, n)
        s = jax.nn.leaky_relu(side @ Wg[i] + bg[i])
        b2 = jax.nn.leaky_relu((ego * side) @ Wb[i] + bb[i])
        ego = s + b2
        ego = ego / (jnp.linalg.norm(ego, axis=1, keepdims=True) + 1e-8)
        embs.append(ego)
    return jnp.mean(jnp.stack(embs, 0), 0)


def _gru(x, wih, whh, bih, bhh):
    def step(h, xt):
        gi = xt @ wih.T + bih
        gh = h @ whh.T + bhh
        ir, iz, inn = jnp.split(gi, 3, axis=-1)
        hr, hz, hn = jnp.split(gh, 3, axis=-1)
        r = jax.nn.sigmoid(ir + hr)
        z = jax.nn.sigmoid(iz + hz)
        ng = jnp.tanh(inn + r * hn)
        hh = (1 - z) * ng + z * h
        return hh, hh
    h0 = jnp.zeros((x.shape[0], x.shape[2]), x.dtype)
    _, ys = jax.lax.scan(step, h0, jnp.swapaxes(x, 0, 1))
    return jnp.swapaxes(ys, 0, 1)


def _ln(x, g, b):
    m = x.mean(-1, keepdims=True)
    v = ((x - m) ** 2).mean(-1, keepdims=True)
    return (x - m) / jnp.sqrt(v + 1e-8) * g + b


def _ssl_pair(a, b):
    a = a / (jnp.linalg.norm(a, axis=-1, keepdims=True) + 1e-8)
    b = b / (jnp.linalg.norm(b, axis=-1, keepdims=True) + 1e-8)
    pos = (a * b).sum(-1)
    neg = (a * jnp.roll(b, 1, axis=0)).sum(-1)
    return -jnp.mean(jax.nn.log_sigmoid(pos - neg))


def _probe_kernel(x_ref, o_ref):
    o_ref[...] = x_ref[...] + 0.0


def kernel(item_emb_w, user_emb_w, cate_emb_w, year_emb_w, month_emb_w, day_emb_w, mu_all_w, sigma_all_w, abs_pos_w, W_gc, b_gc, W_bi, b_bi, W_gc_c, b_gc_c, W_bi_c, b_bi_c, gru_wih, gru_whh, gru_bih, gru_bhh, ln_g, ln_b, ui_vals, uc_vals, it_vals, time_int, user_ids, log_seqs, year, month, day, pos_seqs, neg_seqs, ui_rows, ui_cols, uc_rows, uc_cols, it_rows, it_cols):
    item_w = pl.pallas_call(
        _probe_kernel,
        out_shape=jax.ShapeDtypeStruct(item_emb_w.shape, item_emb_w.dtype),
    )(item_emb_w)
    user_w = user_emb_w
    cate_w = cate_emb_w
    ego_ui = jnp.concatenate([user_w, item_w], 0)
    all_ui = _propagate(ui_rows, ui_cols, ui_vals, ego_ui, W_gc, b_gc, W_bi, b_bi, U + I)
    user_emb, items_emb = all_ui[:U], all_ui[U:]
    ego_uc = jnp.concatenate([user_w, cate_w], 0)
    all_uc = _propagate(uc_rows, uc_cols, uc_vals, ego_uc, W_gc_c, b_gc_c, W_bi_c, b_bi_c, U + C)
    user_emb_c = all_uc[:U]
    con_loss2 = _ssl_pair(user_emb, user_emb_c)
    seqs = items_emb[log_seqs] * (D ** 0.5)
    abs_pos = abs_pos_w[jnp.arange(T)][None, :, :]
    times_emb = jnp.concatenate([year_emb_w, month_emb_w, day_emb_w], 0)
    item_time_embs = _spmm(it_rows, it_cols, it_vals, times_emb, I)
    seqs = seqs + abs_pos + item_time_embs[log_seqs]
    time_embs = month_emb_w[month] + day_emb_w[day]
    hist_t = time_embs[:, :T]
    pred_t = time_embs[:, 1:T + 1]
    mask = log_seqs == 0
    seqs = seqs * (~mask)[:, :, None]
    attn_mask = ~jnp.tril(jnp.ones((T, T), dtype=bool))
    Fu = _gru(seqs, gru_wih, gru_whh, gru_bih, gru_bhh)
    Gu = user_emb[user_ids][:, None, :]
    mu = mu_all_w[user_ids]
    sig = sigma_all_w[user_ids]
    delta_t = time_int[user_ids]
    te = (pred_t[:, None, :, :] - hist_t[:, :, None, :]).sum(-1)
    intent = Fu @ jnp.swapaxes(seqs, 1, 2)
    scores = te * jax.nn.softplus(sig)[:, :, None] + mu[:, :, None] + intent - 0.001 * delta_t[:, :, None]
    scores = jnp.where(attn_mask[None], jnp.full_like(scores, NEG), scores)
    attn = jax.nn.softmax(scores, axis=-1)
    E_recom = _ln(attn @ seqs, ln_g, ln_b)
    con_loss = _ssl_pair(Fu.mean(1), Gu[:, 0, :])
    log_feats = E_recom + _ln(Fu, ln_g, ln_b)
    pos_logits = (log_feats * item_w[pos_seqs]).sum(-1)
    neg_logits = (log_feats * item_w[neg_seqs]).sum(-1)
    return pos_logits, neg_logits, BETA * con_loss + BETA * con_loss2


# speedup vs baseline: 1.7591x; 1.7591x over previous
import functools

import jax
import jax.numpy as jnp
from jax import lax
from jax.experimental import pallas as pl
from jax.experimental.pallas import tpu as pltpu
from jax.experimental.pallas import tpu_sc as plsc

U, I, C = 100000, 50000, 1000
NY, NM, ND = 20, 13, 32
D, T, L = 64, 50, 3
B = 1024
BETA = 0.1
NEG = -2.0 ** 32 + 1


_LN = 16
_NS = 16
_NC = 2
_W = 1024
_GB = 128
_CBUF = _GB + 2 * _LN
_ZR = 16
_RCHUNK = 12288


def _spmm_sc_body(nchunk, R, nwin, ego_hbm, rows_hbm, cols_hbm, vals_hbm,
                  out_hbm, acc, rowb, colb, valb, ccol, cval, clrow, gbuf, zbuf):
    cid = lax.axis_index("c")
    sid = lax.axis_index("s")
    esub = nwin * _W
    rs = R // _NS
    zf = jnp.zeros((_LN,), jnp.float32)
    zi = jnp.zeros((_LN,), jnp.int32)

    @pl.loop(0, _ZR)
    def _(r):
        for cth in range(8):
            zbuf[r, pl.ds(cth * _LN, _LN)] = zf

    @pl.loop(0, _CBUF // _LN)
    def _(g):
        ccol[pl.ds(g * _LN, _LN)] = zi
        clrow[pl.ds(g * _LN, _LN)] = zi
        cval[pl.ds(g * _LN, _LN)] = zf

    def process_block():
        pltpu.sync_copy(ego_hbm.at[ccol.at[pl.ds(0, _GB)]], gbuf)

        @pl.loop(0, _GB)
        def _(j):
            vv = plsc.load_gather(cval, [jnp.full((_LN,), j, jnp.int32)])
            for cth in range(4):
                sl = pl.ds(cth * _LN, _LN)
                gbuf[j, sl] = gbuf[j, sl] * vv

        pltpu.sync_copy(gbuf, acc.at[clrow.at[pl.ds(0, _GB)]], add=True)

    def shift_tail():
        ccol[pl.ds(0, _LN)] = ccol[pl.ds(_GB, _LN)]
        clrow[pl.ds(0, _LN)] = clrow[pl.ds(_GB, _LN)]
        cval[pl.ds(0, _LN)] = cval[pl.ds(_GB, _LN)]

    npass = -(-nchunk // _NC)

    @pl.loop(0, npass)
    def _(p):
        chunk = p * _NC + cid
        base = chunk * R

        @pl.when(chunk < nchunk)
        def _():
            @pl.loop(0, rs // _ZR)
            def _(z):
                pltpu.sync_copy(zbuf, acc.at[pl.ds(sid * rs + z * _ZR, _ZR)])
            plsc.subcore_barrier()

            def window(w, off):
                wbase = sid * esub + w * _W
                pltpu.sync_copy(rows_hbm.at[pl.ds(wbase, _W)], rowb)
                pltpu.sync_copy(cols_hbm.at[pl.ds(wbase, _W)], colb)
                pltpu.sync_copy(vals_hbm.at[pl.ds(wbase, _W)], valb)

                def group(g, off):
                    sl = pl.ds(g * _LN, _LN)
                    rv = rowb[sl]
                    m = (rv >= base) & (rv < base + R)
                    plsc.store_compressed(ccol.at[pl.ds(off, _LN)], colb[sl], mask=m)
                    plsc.store_compressed(cval.at[pl.ds(off, _LN)], valb[sl], mask=m)
                    plsc.store_compressed(clrow.at[pl.ds(off, _LN)], rv - base, mask=m)
                    off = off + jnp.sum(m.astype(jnp.int32))

                    @pl.when(off >= _GB)
                    def _():
                        process_block()
                        shift_tail()

                    return jnp.where(off >= _GB, off - _GB, off)

                return lax.fori_loop(0, _W // _LN, group, off)

            off = lax.fori_loop(0, nwin, window, jnp.int32(0))

            cval[pl.ds(off, _LN)] = zf

            @pl.loop(0, _CBUF // _LN)
            def _(g):
                @pl.when(g * _LN >= off)
                def _():
                    cval[pl.ds(g * _LN, _LN)] = zf

            @pl.when(off > 0)
            def _():
                process_block()

            plsc.subcore_barrier()
            pltpu.sync_copy(acc.at[pl.ds(sid * rs, rs)],
                            out_hbm.at[pl.ds(base + sid * rs, rs)])
            plsc.subcore_barrier()


def _sc_spmm(mat, rows, cols, vals, n_out):
    E = rows.shape[0]
    R = _RCHUNK
    nchunk = -(-n_out // R)
    n_pad = nchunk * R
    esub = -(-E // (_NS * _W)) * _W
    E_pad = esub * _NS
    nwin = esub // _W
    pad = E_pad - E
    if pad:
        rows = jnp.concatenate([rows, jnp.zeros((pad,), rows.dtype)])
        cols = jnp.concatenate([cols, jnp.zeros((pad,), cols.dtype)])
        vals = jnp.concatenate([vals, jnp.zeros((pad,), vals.dtype)])
    mesh = plsc.VectorSubcoreMesh(core_axis_name="c", subcore_axis_name="s",
                                  num_cores=_NC, num_subcores=_NS)
    kern = pl.kernel(
        functools.partial(_spmm_sc_body, nchunk, R, nwin),
        out_type=jax.ShapeDtypeStruct((n_pad, 128), jnp.float32),
        mesh=mesh,
        compiler_params=pltpu.CompilerParams(needs_layout_passes=False),
        scratch_types=[
            pltpu.VMEM_SHARED((R, 128), jnp.float32),
            pltpu.VMEM((_W,), jnp.int32),
            pltpu.VMEM((_W,), jnp.int32),
            pltpu.VMEM((_W,), jnp.float32),
            pltpu.VMEM((_CBUF,), jnp.int32),
            pltpu.VMEM((_CBUF,), jnp.float32),
            pltpu.VMEM((_CBUF,), jnp.int32),
            pltpu.VMEM((_GB, 128), jnp.float32),
            pltpu.VMEM((_ZR, 128), jnp.float32),
        ],
    )
    mat128 = jnp.pad(mat, ((0, 0), (0, 128 - mat.shape[1])))
    out = kern(mat128, rows, cols, vals)
    return out[:n_out, :64]


def _sc_gather(table128, idx, blk):
    M = idx.shape[0]
    mw = M // (_NC * _NS)
    nblk = mw // blk

    def body(tab_hbm, idx_hbm, out_hbm, ivm, gv):
        w = lax.axis_index("c") * _NS + lax.axis_index("s")

        @pl.loop(0, nblk)
        def _(b):
            o = w * mw + b * blk
            pltpu.sync_copy(idx_hbm.at[pl.ds(o, blk)], ivm)
            pltpu.sync_copy(tab_hbm.at[ivm], gv)
            pltpu.sync_copy(gv, out_hbm.at[pl.ds(o, blk)])

    mesh = plsc.VectorSubcoreMesh(core_axis_name="c", subcore_axis_name="s",
                                  num_cores=_NC, num_subcores=_NS)
    kern = pl.kernel(
        body,
        out_type=jax.ShapeDtypeStruct((M, 128), jnp.float32),
        mesh=mesh,
        compiler_params=pltpu.CompilerParams(needs_layout_passes=False),
        scratch_types=[
            pltpu.VMEM((blk,), jnp.int32),
            pltpu.VMEM((blk, 128), jnp.float32),
        ],
    )
    return kern(table128, idx)


_RB = 1000


def _dense_body(final, side_ref, ego_ref, acc_ref, wg_ref, bg_ref, wb_ref,
                bb_ref, ego_o, acc_o):
    side = side_ref[...]
    ego = ego_ref[...]
    s = jax.nn.leaky_relu(
        jnp.dot(side, wg_ref[...], preferred_element_type=jnp.float32)
        + bg_ref[...])
    b2 = jax.nn.leaky_relu(
        jnp.dot(ego * side, wb_ref[...], preferred_element_type=jnp.float32)
        + bb_ref[...])
    e = s + b2
    nrm = jnp.sqrt(jnp.sum(e * e, axis=1, keepdims=True))
    e = e / (nrm + 1e-8)
    ego_o[...] = e
    if final:
        acc_o[...] = (acc_ref[...] + e) * 0.25
    else:
        acc_o[...] = acc_ref[...] + e


def _dense_step(side, ego, acc, Wg, bg, Wb, bb, final):
    n = side.shape[0]
    grid = n // _RB
    rspec = pl.BlockSpec((_RB, D), lambda i: (i, 0))
    wspec = pl.BlockSpec((D, D), lambda i: (0, 0))
    bspec = pl.BlockSpec((1, D), lambda i: (0, 0))
    return pl.pallas_call(
        functools.partial(_dense_body, final),
        grid=(grid,),
        in_specs=[rspec, rspec, rspec, wspec, bspec, wspec, bspec],
        out_specs=[rspec, rspec],
        out_shape=[jax.ShapeDtypeStruct((n, D), jnp.float32),
                   jax.ShapeDtypeStruct((n, D), jnp.float32)],
    )(side, ego, acc, Wg, bg.reshape(1, D), Wb, bb.reshape(1, D))


def _propagate(rows, cols, vals, ego, Wg, bg, Wb, bb, n):
    acc = ego
    for i in range(L):
        side = _sc_spmm(ego, rows, cols, vals, n)
        ego, acc = _dense_step(side, ego, acc, Wg[i], bg[i], Wb[i], bb[i],
                               final=(i == L - 1))
    return acc


_IB = 400


def _seqtab_body(c0, c1, c2, v0, v1, v2, te_ref, it_ref, out_ref):
    res = 8.0 * it_ref[...]
    nt = te_ref.shape[0]
    for ck, vk in ((c0, v0), (c1, v1), (c2, v2)):
        c = ck[0, 0, :]
        v = vk[0, 0, :]
        oh = jnp.where(
            c[:, None] == lax.broadcasted_iota(jnp.int32, (_IB, nt), 1),
            v[:, None], 0.0)
        res = res + jnp.dot(oh, te_ref[...], preferred_element_type=jnp.float32)
    out_ref[...] = jnp.concatenate(
        [res, jnp.zeros((_IB, 64), jnp.float32)], axis=1)


def _seq_table(it_cols, it_vals, times_emb, items_emb):
    nt = 72
    te = jnp.pad(times_emb, ((0, nt - times_emb.shape[0]), (0, 0)))
    c = it_cols.reshape(I, 3)
    v = it_vals.reshape(I, 3)
    nb = I // _IB
    cs = [c[:, k].reshape(nb, 1, _IB) for k in range(3)]
    vs = [v[:, k].reshape(nb, 1, _IB) for k in range(3)]
    ispec = pl.BlockSpec((1, 1, _IB), lambda i: (i, 0, 0))
    return pl.pallas_call(
        _seqtab_body,
        grid=(nb,),
        in_specs=[ispec, ispec, ispec, ispec, ispec, ispec,
                  pl.BlockSpec((nt, D), lambda i: (0, 0)),
                  pl.BlockSpec((_IB, D), lambda i: (i, 0))],
        out_specs=pl.BlockSpec((_IB, 128), lambda i: (i, 0)),
        out_shape=jax.ShapeDtypeStruct((I, 128), jnp.float32),
    )(cs[0], cs[1], cs[2], vs[0], vs[1], vs[2], te, items_emb)


_GBB = 128


def _gru_body(sq_ref, ls_ref, pos_ref, wir, wiz, win, whr, whz, whn,
              br, bz, bni, bnh, fu_ref, sm_ref):
    mk = jnp.where(ls_ref[...] == 0, 0.0, 1.0)
    h = jnp.zeros((_GBB, D), jnp.float32)
    for t in range(T):
        x = (sq_ref[:, t, :] + pos_ref[t, :][None, :]) * mk[:, t][:, None]
        gr = jax.nn.sigmoid(
            jnp.dot(x, wir[...], preferred_element_type=jnp.float32)
            + jnp.dot(h, whr[...], preferred_element_type=jnp.float32)
            + br[...])
        gz = jax.nn.sigmoid(
            jnp.dot(x, wiz[...], preferred_element_type=jnp.float32)
            + jnp.dot(h, whz[...], preferred_element_type=jnp.float32)
            + bz[...])
        ng = jnp.tanh(
            jnp.dot(x, win[...], preferred_element_type=jnp.float32) + bni[...]
            + gr * (jnp.dot(h, whn[...], preferred_element_type=jnp.float32)
                    + bnh[...]))
        h = (1.0 - gz) * ng + gz * h
        fu_ref[:, t, :] = h
        sm_ref[:, t, :] = x


def _gru_run(seqs_g, log_seqs, abs_pos, wih, whh, bih, bhh):
    wir, wiz, win = [wih[k * D:(k + 1) * D].T for k in range(3)]
    whr, whz, whn = [whh[k * D:(k + 1) * D].T for k in range(3)]
    br = (bih[0:D] + bhh[0:D]).reshape(1, D)
    bz = (bih[D:2 * D] + bhh[D:2 * D]).reshape(1, D)
    bni = bih[2 * D:].reshape(1, D)
    bnh = bhh[2 * D:].reshape(1, D)
    grid = B // _GBB
    sspec = pl.BlockSpec((_GBB, T, D), lambda i: (i, 0, 0))
    wspec = pl.BlockSpec((D, D), lambda i: (0, 0))
    bspec = pl.BlockSpec((1, D), lambda i: (0, 0))
    return pl.pallas_call(
        _gru_body,
        grid=(grid,),
        in_specs=[sspec, pl.BlockSpec((_GBB, T), lambda i: (i, 0)),
                  pl.BlockSpec((T, D), lambda i: (0, 0)),
                  wspec, wspec, wspec, wspec, wspec, wspec,
                  bspec, bspec, bspec, bspec],
        out_specs=[sspec, sspec],
        out_shape=[jax.ShapeDtypeStruct((B, T, D), jnp.float32),
                   jax.ShapeDtypeStruct((B, T, D), jnp.float32)],
    )(seqs_g, log_seqs, abs_pos, wir, wiz, win, whr, whz, whn,
      br, bz, bni, bnh)


def _ln_in(x, g, b):
    m = jnp.mean(x, axis=-1, keepdims=True)
    v = jnp.mean((x - m) ** 2, axis=-1, keepdims=True)
    return (x - m) / jnp.sqrt(v + 1e-8) * g + b


def _attn_body(fu_ref, sm_ref, mh_ref, mp_ref, dh_ref, dp_ref, mu_ref,
               sig_ref, dt_ref, pw_ref, nw_ref, mw_ref, dw_ref, g_ref, b_ref,
               pos_o, neg_o, fm_o):
    msum = jnp.sum(mw_ref[...], axis=1, keepdims=True)
    dsum = jnp.sum(dw_ref[...], axis=1, keepdims=True)

    def tsum(mv, dv):
        s = jnp.zeros(mv.shape, jnp.float32)
        for k in range(NM):
            s = s + jnp.where(mv == k, msum[k, 0], 0.0)
        for k in range(ND):
            s = s + jnp.where(dv == k, dsum[k, 0], 0.0)
        return s

    SH = tsum(mh_ref[...], dh_ref[...])
    SP = tsum(mp_ref[...], dp_ref[...])
    fu = fu_ref[...]
    sm = sm_ref[...]
    te3 = SP[:, None, :] - SH[:, :, None]
    intent = lax.dot_general(fu, sm, (((2,), (2,)), ((0,), (0,))),
                             preferred_element_type=jnp.float32)
    sp_s = jax.nn.softplus(sig_ref[...])
    sc3 = (te3 * sp_s[:, :, None] + mu_ref[...][:, :, None] + intent
           - 0.001 * dt_ref[...][:, :, None])
    ii = lax.broadcasted_iota(jnp.int32, (T, T), 0)
    jj = lax.broadcasted_iota(jnp.int32, (T, T), 1)
    sc3 = jnp.where((jj > ii)[None, :, :], jnp.float32(NEG), sc3)
    mx = jnp.max(sc3, axis=-1, keepdims=True)
    ex = jnp.exp(sc3 - mx)
    attn = ex / jnp.sum(ex, axis=-1, keepdims=True)
    E = lax.dot_general(attn, sm, (((2,), (1,)), ((0,), (0,))),
                        preferred_element_type=jnp.float32)
    g = g_ref[...]
    b = b_ref[...]
    logf = _ln_in(E, g, b) + _ln_in(fu, g, b)
    pos_o[...] = jnp.sum(logf * pw_ref[...], axis=-1)
    neg_o[...] = jnp.sum(logf * nw_ref[...], axis=-1)
    fm_o[...] = jnp.mean(fu, axis=1)


def _attn_run(Fu, seqm, mh, mp, dh, dp, mu, sig, dt, pw, nw, month_w, day_w,
              ln_g, ln_b):
    grid = B // _GBB
    s3 = pl.BlockSpec((_GBB, T, D), lambda i: (i, 0, 0))
    s2 = pl.BlockSpec((_GBB, T), lambda i: (i, 0))
    mwp = jnp.pad(month_w, ((0, 16 - NM), (0, 0)))
    return pl.pallas_call(
        _attn_body,
        grid=(grid,),
        in_specs=[s3, s3, s2, s2, s2, s2, s2, s2, s2, s3, s3,
                  pl.BlockSpec((16, D), lambda i: (0, 0)),
                  pl.BlockSpec((ND, D), lambda i: (0, 0)),
                  pl.BlockSpec((1, D), lambda i: (0, 0)),
                  pl.BlockSpec((1, D), lambda i: (0, 0))],
        out_specs=[s2, s2, pl.BlockSpec((_GBB, D), lambda i: (i, 0))],
        out_shape=[jax.ShapeDtypeStruct((B, T), jnp.float32),
                   jax.ShapeDtypeStruct((B, T), jnp.float32),
                   jax.ShapeDtypeStruct((B, D), jnp.float32)],
    )(Fu, seqm, mh, mp, dh, dp, mu, sig, dt, pw, nw, mwp, day_w,
      ln_g.reshape(1, D), ln_b.reshape(1, D))


def _ssl_body(a_ref, b_ref, br_ref, out_ref):
    @pl.when(pl.program_id(0) == 0)
    def _():
        out_ref[0, 0] = 0.0

    a = a_ref[...]
    bb = b_ref[...]
    br = br_ref[...]
    an = a / (jnp.sqrt(jnp.sum(a * a, 1, keepdims=True)) + 1e-8)
    bn = bb / (jnp.sqrt(jnp.sum(bb * bb, 1, keepdims=True)) + 1e-8)
    brn = br / (jnp.sqrt(jnp.sum(br * br, 1, keepdims=True)) + 1e-8)
    pos = jnp.sum(an * bn, axis=1)
    neg = jnp.sum(an * brn, axis=1)
    out_ref[0, 0] += jnp.sum(jax.nn.log_sigmoid(pos - neg))


def _ssl_loss(a, b, rb):
    n = a.shape[0]
    blk = _RB if n % _RB == 0 else n
    grid = n // blk
    rspec = pl.BlockSpec((blk, D), lambda i: (i, 0))
    tot = pl.pallas_call(
        _ssl_body,
        grid=(grid,),
        in_specs=[rspec, rspec, rspec],
        out_specs=pl.BlockSpec(memory_space=pltpu.SMEM),
        out_shape=jax.ShapeDtypeStruct((1, 1), jnp.float32),
    )(a, b, rb)
    return -tot[0, 0] / n


def kernel(item_emb_w, user_emb_w, cate_emb_w, year_emb_w, month_emb_w, day_emb_w, mu_all_w, sigma_all_w, abs_pos_w, W_gc, b_gc, W_bi, b_bi, W_gc_c, b_gc_c, W_bi_c, b_bi_c, gru_wih, gru_whh, gru_bih, gru_bhh, ln_g, ln_b, ui_vals, uc_vals, it_vals, time_int, user_ids, log_seqs, year, month, day, pos_seqs, neg_seqs, ui_rows, ui_cols, uc_rows, uc_cols, it_rows, it_cols):
    ego_ui = jnp.concatenate([user_emb_w, item_emb_w], 0)
    all_ui = _propagate(ui_rows, ui_cols, ui_vals, ego_ui, W_gc, b_gc,
                        W_bi, b_bi, U + I)
    user_emb, items_emb = all_ui[:U], all_ui[U:]
    ego_uc = jnp.concatenate([user_emb_w, cate_emb_w], 0)
    all_uc = _propagate(uc_rows, uc_cols, uc_vals, ego_uc, W_gc_c, b_gc_c,
                        W_bi_c, b_bi_c, U + C)
    user_emb_c = all_uc[:U]
    con_loss2 = _ssl_loss(user_emb, user_emb_c,
                          jnp.roll(user_emb_c, 1, axis=0))

    times_emb = jnp.concatenate([year_emb_w, month_emb_w, day_emb_w], 0)
    seq_table = _seq_table(it_cols, it_vals, times_emb, items_emb)
    seqs_g = _sc_gather(seq_table, log_seqs.reshape(-1), 320)
    seqs_g = seqs_g[:, :D].reshape(B, T, D)

    item128 = jnp.pad(item_emb_w, ((0, 0), (0, 64)))
    pn = _sc_gather(item128,
                    jnp.concatenate([pos_seqs.reshape(-1),
                                     neg_seqs.reshape(-1)]), 320)
    pw = pn[:B * T, :D].reshape(B, T, D)
    nw = pn[B * T:, :D].reshape(B, T, D)

    user128 = jnp.pad(user_emb, ((0, 0), (0, 64)))
    mu128 = jnp.pad(mu_all_w, ((0, 0), (0, 128 - T)))
    sig128 = jnp.pad(sigma_all_w, ((0, 0), (0, 128 - T)))
    ti128 = jnp.pad(time_int, ((0, 0), (0, 128 - T)))
    gu = _sc_gather(user128, user_ids, 32)[:, :D]
    mu = _sc_gather(mu128, user_ids, 32)[:, :T]
    sig = _sc_gather(sig128, user_ids, 32)[:, :T]
    dt = _sc_gather(ti128, user_ids, 32)[:, :T]

    Fu, seqm = _gru_run(seqs_g, log_seqs, abs_pos_w, gru_wih, gru_whh,
                        gru_bih, gru_bhh)

    mh, mp = month[:, :T], month[:, 1:T + 1]
    dh, dp = day[:, :T], day[:, 1:T + 1]
    pos_logits, neg_logits, fmean = _attn_run(
        Fu, seqm, mh, mp, dh, dp, mu, sig, dt, pw, nw,
        month_emb_w, day_emb_w, ln_g, ln_b)

    con_loss = _ssl_loss(fmean, gu, jnp.roll(gu, 1, axis=0))
    return pos_logits, neg_logits, BETA * con_loss + BETA * con_loss2
